# Initial kernel scaffold; baseline (speedup 1.0000x reference)
import jax
import jax.numpy as jnp
from jax.experimental import pallas as pl


def kernel(lig_h0, lig_x0, kp_h0, kp_x0, kp_v0, timestep, lig_batch_idx, kp_batch_idx, ll_edge_index, kl_src, kl_dst, params):
    raise NotImplementedError("write your pallas kernel here")


# speedup vs baseline: 8.9865x; 8.9865x over previous
import functools
import jax
import jax.numpy as jnp
from jax.experimental import pallas as pl
from jax.experimental.pallas import tpu as pltpu

H = 64
V = 16
B = 250
NLIG = 50000
NKP = 5000
NP_L = 50176
NP_K = 5120
E_LL = 800000
E_KL = 400000
EP_LL = 819200
EP_KL = 409600
EB = 1024
NB = 256
MW = 112


def _silu(x):
    return x * jax.nn.sigmoid(x)


def _gvp(s, vx, vy, vz, w, gate):
    Wh, Wu, Wfs, Wfh, bf = w
    vhx = jnp.dot(vx, Wh, preferred_element_type=jnp.float32)
    vhy = jnp.dot(vy, Wh, preferred_element_type=jnp.float32)
    vhz = jnp.dot(vz, Wh, preferred_element_type=jnp.float32)
    sh = jnp.sqrt(vhx * vhx + vhy * vhy + vhz * vhz + 1e-8)
    s_out = _silu(jnp.dot(s, Wfs, preferred_element_type=jnp.float32)
                  + jnp.dot(sh, Wfh, preferred_element_type=jnp.float32) + bf)
    vux = jnp.dot(vhx, Wu, preferred_element_type=jnp.float32)
    vuy = jnp.dot(vhy, Wu, preferred_element_type=jnp.float32)
    vuz = jnp.dot(vhz, Wu, preferred_element_type=jnp.float32)
    if gate:
        n = jnp.sqrt(vux * vux + vuy * vuy + vuz * vuz + 1e-8)
        g = jax.nn.sigmoid(n)
        vux = g * vux
        vuy = g * vuy
        vuz = g * vuz
    return s_out, vux, vuy, vuz


def _flatten_chain(chain):
    flat = []
    for p in chain:
        s_in = p['Wf'].shape[0] - p['Wh'].shape[1]
        flat += [p['Wh'], p['Wu'], p['Wf'][:s_in], p['Wf'][s_in:],
                 p['bf'].reshape(1, -1)]
    return flat


def _unpack_ws(refs, n_gvp):
    ws = []
    for i in range(n_gvp):
        ws.append(tuple(r[...] for r in refs[5 * i:5 * i + 5]))
    return ws


def _rep_spec(a):
    return pl.BlockSpec(a.shape, lambda i: (0,) * a.ndim)


def _enc_body(n_extra, h_ref, bi_ref, x16_ref, ts_ref, w_ref, wt_ref, b_ref,
              g_ref, be_ref, out_ref, *, vxyz_ref=None):
    bi = bi_ref[...]
    iota = jax.lax.broadcasted_iota(jnp.int32, (NB, 256), 1)
    oh = (iota == bi).astype(jnp.float32)
    t = jnp.dot(oh, ts_ref[...], preferred_element_type=jnp.float32)
    s = jnp.dot(h_ref[...], w_ref[...], preferred_element_type=jnp.float32)
    s = _silu(s + t * wt_ref[...] + b_ref[...])
    mu = jnp.mean(s, axis=-1, keepdims=True)
    var = jnp.mean((s - mu) ** 2, axis=-1, keepdims=True)
    s = (s - mu) / jnp.sqrt(var + 1e-5) * g_ref[...] + be_ref[...]
    if vxyz_ref is None:
        vpart = jnp.zeros((NB, 48), jnp.float32)
    else:
        vpart = vxyz_ref[...]
    out_ref[...] = jnp.concatenate([s, vpart, x16_ref[...]], axis=1)


def _encode(h0p, bip, x16, ts_pad, W, b, g, be, vxyzp, npad):
    nblk = npad // NB
    Wh0 = W[:-1]
    wt = W[-1:].reshape(1, H)
    ins = [h0p, bip, x16, ts_pad, Wh0, wt, b.reshape(1, H), g.reshape(1, H),
           be.reshape(1, H)]
    specs = [
        pl.BlockSpec((NB, h0p.shape[1]), lambda i: (i, 0)),
        pl.BlockSpec((NB, 1), lambda i: (i, 0)),
        pl.BlockSpec((NB, 16), lambda i: (i, 0)),
        _rep_spec(ts_pad), _rep_spec(Wh0), _rep_spec(wt), _rep_spec(b.reshape(1, H)),
        _rep_spec(g.reshape(1, H)), _rep_spec(be.reshape(1, H)),
    ]
    if vxyzp is not None:
        ins.append(vxyzp)
        specs.append(pl.BlockSpec((NB, 48), lambda i: (i, 0)))

        def body(h, bi, x16r, tsr, wr, wtr, br, gr, ber, vr, o):
            _enc_body(0, h, bi, x16r, tsr, wr, wtr, br, gr, ber, o, vxyz_ref=vr)
    else:
        def body(h, bi, x16r, tsr, wr, wtr, br, gr, ber, o):
            _enc_body(0, h, bi, x16r, tsr, wr, wtr, br, gr, ber, o)
    return pl.pallas_call(
        body,
        grid=(nblk,),
        in_specs=specs,
        out_specs=pl.BlockSpec((NB, 128), lambda i: (i, 0)),
        out_shape=jax.ShapeDtypeStruct((npad, 128), jnp.float32),
    )(*ins)


def _msg_body(src_ref, dst_ref, *rest):
    out_ref = rest[-1]
    ws = _unpack_ws(rest[:-1], 3)
    src = src_ref[...]
    dst = dst_ref[...]
    s = jnp.concatenate([src[:, :64], dst[:, :64]], axis=1)
    dx = dst[:, 112:113] - src[:, 112:113]
    dy = dst[:, 113:114] - src[:, 113:114]
    dz = dst[:, 114:115] - src[:, 114:115]
    inv = 1.0 / (jnp.sqrt(dx * dx + dy * dy + dz * dz) + 1e-8)
    vx = jnp.concatenate([src[:, 64:80], dst[:, 64:80], dx * inv], axis=1)
    vy = jnp.concatenate([src[:, 80:96], dst[:, 80:96], dy * inv], axis=1)
    vz = jnp.concatenate([src[:, 96:112], dst[:, 96:112], dz * inv], axis=1)
    for w in ws:
        s, vx, vy, vz = _gvp(s, vx, vy, vz, w, gate=True)
    out_ref[...] = jnp.concatenate([s, vx, vy, vz], axis=1)


def _messages(src_rows, dst_rows, chain):
    ep = src_rows.shape[0]
    flat = _flatten_chain(chain)
    ins = [src_rows, dst_rows] + flat
    specs = [pl.BlockSpec((EB, 128), lambda i: (i, 0)),
             pl.BlockSpec((EB, 128), lambda i: (i, 0))] + [_rep_spec(a) for a in flat]
    return pl.pallas_call(
        _msg_body,
        grid=(ep // EB,),
        in_specs=specs,
        out_specs=pl.BlockSpec((EB, MW), lambda i: (i, 0)),
        out_shape=jax.ShapeDtypeStruct((ep, MW), jnp.float32),
    )(*ins)


def _upd_body(tab_ref, agg_ref, *rest):
    out_ref = rest[-1]
    g_ref, b_ref = rest[-3], rest[-2]
    ws = _unpack_ws(rest[:-3], 2)
    tab = tab_ref[...]
    agg = agg_ref[...]
    s0 = tab[:, :64]
    vx0, vy0, vz0 = tab[:, 64:80], tab[:, 80:96], tab[:, 96:112]
    s = jnp.concatenate([s0, agg[:, :64]], axis=1)
    vx = jnp.concatenate([vx0, agg[:, 64:80]], axis=1)
    vy = jnp.concatenate([vy0, agg[:, 80:96]], axis=1)
    vz = jnp.concatenate([vz0, agg[:, 96:112]], axis=1)
    for w in ws:
        s, vx, vy, vz = _gvp(s, vx, vy, vz, w, gate=True)
    s = s0 + s
    mu = jnp.mean(s, axis=-1, keepdims=True)
    var = jnp.mean((s - mu) ** 2, axis=-1, keepdims=True)
    s = (s - mu) / jnp.sqrt(var + 1e-5) * g_ref[...] + b_ref[...]
    out_ref[...] = jnp.concatenate(
        [s, vx0 + vx, vy0 + vy, vz0 + vz, tab[:, 112:128]], axis=1)


def _update(table, agg, chain, ln_g, ln_b):
    npad = table.shape[0]
    flat = _flatten_chain(chain)
    ins = [table, agg] + flat + [ln_g.reshape(1, H), ln_b.reshape(1, H)]
    specs = ([pl.BlockSpec((NB, 128), lambda i: (i, 0)),
              pl.BlockSpec((NB, MW), lambda i: (i, 0))]
             + [_rep_spec(a) for a in flat]
             + [_rep_spec(ln_g.reshape(1, H)), _rep_spec(ln_b.reshape(1, H))])
    return pl.pallas_call(
        _upd_body,
        grid=(npad // NB,),
        in_specs=specs,
        out_specs=pl.BlockSpec((NB, 128), lambda i: (i, 0)),
        out_shape=jax.ShapeDtypeStruct((npad, 128), jnp.float32),
    )(*ins)


def _noise_body(tab_ref, *rest):
    eps_ref, v_ref = rest[-2], rest[-1]
    ow_ref, ob_ref = rest[-4], rest[-3]
    ws = _unpack_ws(rest[:-4], 3)
    tab = tab_ref[...]
    s = tab[:, :64]
    vx, vy, vz = tab[:, 64:80], tab[:, 80:96], tab[:, 96:112]
    s, vx, vy, vz = _gvp(s, vx, vy, vz, ws[0], gate=True)
    s, vx, vy, vz = _gvp(s, vx, vy, vz, ws[1], gate=True)
    s, vx, vy, vz = _gvp(s, vx, vy, vz, ws[2], gate=False)
    eps_ref[...] = jnp.dot(s, ow_ref[...], preferred_element_type=jnp.float32) + ob_ref[...]
    v_ref[...] = jnp.concatenate(
        [vx, vy, vz, jnp.zeros((NB, 5), jnp.float32)], axis=1)


def _noise_head(table, noise_chain, out_W, out_b):
    npad = table.shape[0]
    flat = _flatten_chain(noise_chain)
    ins = [table] + flat + [out_W, out_b.reshape(1, -1)]
    specs = ([pl.BlockSpec((NB, 128), lambda i: (i, 0))]
             + [_rep_spec(a) for a in flat]
             + [_rep_spec(out_W), _rep_spec(out_b.reshape(1, -1))])
    return pl.pallas_call(
        _noise_body,
        grid=(npad // NB,),
        in_specs=specs,
        out_specs=[pl.BlockSpec((NB, 64), lambda i: (i, 0)),
                   pl.BlockSpec((NB, 8), lambda i: (i, 0))],
        out_shape=[jax.ShapeDtypeStruct((npad, 64), jnp.float32),
                   jax.ShapeDtypeStruct((npad, 8), jnp.float32)],
    )(*ins)


SC_NC = 2
SC_NS = 16
SC_CH = 128
WIN = 16384
NW_SC = 4
NP_SC = WIN * NW_SC


def _sc_gather(table, idx):
    from jax.experimental.pallas import tpu_sc as plsc
    ep = idx.shape[0]
    per_w = ep // (SC_NC * SC_NS)
    iters = per_w // SC_CH
    mesh = plsc.VectorSubcoreMesh(core_axis_name="c", subcore_axis_name="s")

    @functools.partial(
        pl.kernel, mesh=mesh,
        out_type=jax.ShapeDtypeStruct((ep, 128), jnp.float32),
        scratch_types=[
            pltpu.VMEM((SC_CH,), jnp.int32),
            pltpu.VMEM((SC_CH, 128), jnp.float32),
            pltpu.SemaphoreType.DMA,
        ],
    )
    def gk(table_hbm, idx_hbm, out_hbm, idx_v, rows_v, sem):
        wid = jax.lax.axis_index("s") * SC_NC + jax.lax.axis_index("c")
        base0 = wid * per_w

        def body(i, _):
            base = base0 + i * SC_CH
            pltpu.sync_copy(idx_hbm.at[pl.ds(base, SC_CH)], idx_v)
            pltpu.async_copy(table_hbm.at[idx_v], rows_v, sem).wait()
            pltpu.sync_copy(rows_v, out_hbm.at[pl.ds(base, SC_CH)])
            return 0

        jax.lax.fori_loop(0, iters, body, 0)

    return gk(table, idx)


def _sc_scatter(msg_ll, dstm_ll, msg_kl, dstm_kl, zeros_win):
    from jax.experimental.pallas import tpu_sc as plsc
    mesh = plsc.VectorSubcoreMesh(core_axis_name="c", subcore_axis_name="s")
    per_ll = EP_LL // SC_NS
    per_kl = EP_KL // SC_NS
    rows_w = WIN // SC_NS

    @functools.partial(
        pl.kernel, mesh=mesh,
        out_type=jax.ShapeDtypeStruct((NP_SC, MW), jnp.float32),
        scratch_types=[
            pltpu.VMEM_SHARED((WIN + 8, MW), jnp.float32),
            pltpu.VMEM((SC_CH,), jnp.int32),
            pltpu.VMEM((SC_CH,), jnp.int32),
            pltpu.VMEM((SC_CH, MW), jnp.float32),
        ],
    )
    def sk(mll_hbm, dll_hbm, mkl_hbm, dkl_hbm, z_hbm, out_hbm,
           acc, idx_raw, idx_v, msg_v):
        c = jax.lax.axis_index("c")
        s = jax.lax.axis_index("s")

        def scan_edges(d_hbm, m_hbm, per_sub, w_lo):
            def body(i, _):
                base = s * per_sub + i * SC_CH
                pltpu.sync_copy(d_hbm.at[pl.ds(base, SC_CH)], idx_raw)
                for j in range(SC_CH // 16):
                    sl = pl.ds(j * 16, 16)
                    local = idx_raw[sl] - w_lo
                    ok = (local >= 0) & (local < WIN)
                    idx_v[sl] = jnp.where(ok, local, WIN)
                pltpu.sync_copy(m_hbm.at[pl.ds(base, SC_CH)], msg_v)
                pltpu.sync_copy(msg_v, acc.at[idx_v], add=True)
                return 0

            jax.lax.fori_loop(0, per_sub // SC_CH, body, 0)

        for k in range(NW_SC // SC_NC):
            w = c + SC_NC * k
            w_lo = w * WIN
            pltpu.sync_copy(z_hbm.at[pl.ds(s * rows_w, rows_w)],
                            acc.at[pl.ds(s * rows_w, rows_w)])
            plsc.subcore_barrier()
            scan_edges(dll_hbm, mll_hbm, per_ll, w_lo)
            scan_edges(dkl_hbm, mkl_hbm, per_kl, w_lo)
            plsc.subcore_barrier()
            pltpu.sync_copy(acc.at[pl.ds(s * rows_w, rows_w)],
                            out_hbm.at[pl.ds(w_lo + s * rows_w, rows_w)])
            plsc.subcore_barrier()

    return sk(msg_ll, dstm_ll, msg_kl, dstm_kl, zeros_win)


def _gather_rows(table, idx):
    return jnp.take(table, idx, axis=0)


def _segment_add(msg_ll, dst_ll, msg_kl, dst_kl, npad):
    agg = jnp.zeros((npad, MW), jnp.float32)
    agg = agg.at[dst_ll].add(msg_ll[:E_LL])
    agg = agg.at[dst_kl].add(msg_kl[:E_KL])
    return agg


def _pad_rows(a, npad):
    return jnp.concatenate(
        [a, jnp.zeros((npad - a.shape[0],) + a.shape[1:], a.dtype)], axis=0)


def kernel(lig_h0, lig_x0, kp_h0, kp_x0, kp_v0, timestep, lig_batch_idx,
           kp_batch_idx, ll_edge_index, kl_src, kl_dst, params):
    ts_pad = jnp.concatenate(
        [timestep, jnp.zeros((256 - timestep.shape[0],), jnp.float32)]).reshape(256, 1)

    lig_x16 = _pad_rows(jnp.pad(lig_x0, ((0, 0), (0, 13))), NP_L)
    kp_x16 = _pad_rows(jnp.pad(kp_x0, ((0, 0), (0, 13))), NP_K)
    kp_vxyz = _pad_rows(
        jnp.concatenate([kp_v0[..., 0], kp_v0[..., 1], kp_v0[..., 2]], axis=1), NP_K)

    lig_tab = _encode(_pad_rows(lig_h0, NP_L),
                      _pad_rows(lig_batch_idx.reshape(-1, 1), NP_L), lig_x16,
                      ts_pad, params['lig_enc_W'], params['lig_enc_b'],
                      params['lig_ln_g'], params['lig_ln_b'], None, NP_L)
    kp_tab = _encode(_pad_rows(kp_h0, NP_K),
                     _pad_rows(kp_batch_idx.reshape(-1, 1), NP_K), kp_x16,
                     ts_pad, params['kp_enc_W'], params['kp_enc_b'],
                     params['kp_ln_g'], params['kp_ln_b'], kp_vxyz, NP_K)

    ll_src = jnp.concatenate([ll_edge_index[0], jnp.zeros((EP_LL - E_LL,), jnp.int32)])
    ll_dst = jnp.concatenate([ll_edge_index[1], jnp.zeros((EP_LL - E_LL,), jnp.int32)])
    kl_srcp = jnp.concatenate([kl_src, jnp.zeros((EP_KL - E_KL,), jnp.int32)])
    kl_dstp = jnp.concatenate([kl_dst, jnp.zeros((EP_KL - E_KL,), jnp.int32)])

    for lp in params['convs']:
        src_rows = _gather_rows(lig_tab, ll_src)
        dst_rows = _gather_rows(lig_tab, ll_dst)
        msg_ll = _messages(src_rows, dst_rows, lp['ll_msg'])
        src2 = _gather_rows(kp_tab, kl_srcp)
        dst2 = _gather_rows(lig_tab, kl_dstp)
        msg_kl = _messages(src2, dst2, lp['kl_msg'])
        agg = _segment_add(msg_ll, ll_edge_index[1], msg_kl, kl_dst, NP_L)
        lig_tab = _update(lig_tab, agg, lp['upd'], lp['ln_g'], lp['ln_b'])

    eps_p, v_p = _noise_head(lig_tab, params['noise'], params['out_W'],
                             params['out_b'])
    eps_h = eps_p[:NLIG]
    v = jnp.stack([v_p[:NLIG, 0], v_p[:NLIG, 1], v_p[:NLIG, 2]], axis=1).reshape(NLIG, 1, 3)
    return eps_h, v


# speedup vs baseline: 12.3232x; 1.3713x over previous
import functools
import jax
import jax.numpy as jnp
from jax.experimental import pallas as pl
from jax.experimental.pallas import tpu as pltpu

H = 64
V = 16
B = 250
NLIG = 50000
NKP = 5000
NP_L = 50176
NP_K = 5120
E_LL = 800000
E_KL = 400000
EP_LL = 819200
EP_KL = 409600
EB = 1024
NB = 256
MW = 128


def _silu(x):
    return x * jax.nn.sigmoid(x)


def _gvp(s, vx, vy, vz, w, gate):
    Wh, Wu, Wfs, Wfh, bf = w
    vhx = jnp.dot(vx, Wh, preferred_element_type=jnp.float32)
    vhy = jnp.dot(vy, Wh, preferred_element_type=jnp.float32)
    vhz = jnp.dot(vz, Wh, preferred_element_type=jnp.float32)
    sh = jnp.sqrt(vhx * vhx + vhy * vhy + vhz * vhz + 1e-8)
    s_out = _silu(jnp.dot(s, Wfs, preferred_element_type=jnp.float32)
                  + jnp.dot(sh, Wfh, preferred_element_type=jnp.float32) + bf)
    vux = jnp.dot(vhx, Wu, preferred_element_type=jnp.float32)
    vuy = jnp.dot(vhy, Wu, preferred_element_type=jnp.float32)
    vuz = jnp.dot(vhz, Wu, preferred_element_type=jnp.float32)
    if gate:
        n = jnp.sqrt(vux * vux + vuy * vuy + vuz * vuz + 1e-8)
        g = jax.nn.sigmoid(n)
        vux = g * vux
        vuy = g * vuy
        vuz = g * vuz
    return s_out, vux, vuy, vuz


def _flatten_chain(chain):
    flat = []
    for p in chain:
        s_in = p['Wf'].shape[0] - p['Wh'].shape[1]
        flat += [p['Wh'], p['Wu'], p['Wf'][:s_in], p['Wf'][s_in:],
                 p['bf'].reshape(1, -1)]
    return flat


def _unpack_ws(refs, n_gvp):
    ws = []
    for i in range(n_gvp):
        ws.append(tuple(r[...] for r in refs[5 * i:5 * i + 5]))
    return ws


def _rep_spec(a):
    return pl.BlockSpec(a.shape, lambda i: (0,) * a.ndim)


def _enc_body(n_extra, h_ref, bi_ref, x16_ref, ts_ref, w_ref, wt_ref, b_ref,
              g_ref, be_ref, out_ref, *, vxyz_ref=None):
    bi = bi_ref[...]
    iota = jax.lax.broadcasted_iota(jnp.int32, (NB, 256), 1)
    oh = (iota == bi).astype(jnp.float32)
    t = jnp.dot(oh, ts_ref[...], preferred_element_type=jnp.float32)
    s = jnp.dot(h_ref[...], w_ref[...], preferred_element_type=jnp.float32)
    s = _silu(s + t * wt_ref[...] + b_ref[...])
    mu = jnp.mean(s, axis=-1, keepdims=True)
    var = jnp.mean((s - mu) ** 2, axis=-1, keepdims=True)
    s = (s - mu) / jnp.sqrt(var + 1e-5) * g_ref[...] + be_ref[...]
    if vxyz_ref is None:
        vpart = jnp.zeros((NB, 48), jnp.float32)
    else:
        vpart = vxyz_ref[...]
    out_ref[...] = jnp.concatenate([s, vpart, x16_ref[...]], axis=1)


def _encode(h0p, bip, x16, ts_pad, W, b, g, be, vxyzp, npad):
    nblk = npad // NB
    Wh0 = W[:-1]
    wt = W[-1:].reshape(1, H)
    ins = [h0p, bip, x16, ts_pad, Wh0, wt, b.reshape(1, H), g.reshape(1, H),
           be.reshape(1, H)]
    specs = [
        pl.BlockSpec((NB, h0p.shape[1]), lambda i: (i, 0)),
        pl.BlockSpec((NB, 1), lambda i: (i, 0)),
        pl.BlockSpec((NB, 16), lambda i: (i, 0)),
        _rep_spec(ts_pad), _rep_spec(Wh0), _rep_spec(wt), _rep_spec(b.reshape(1, H)),
        _rep_spec(g.reshape(1, H)), _rep_spec(be.reshape(1, H)),
    ]
    if vxyzp is not None:
        ins.append(vxyzp)
        specs.append(pl.BlockSpec((NB, 48), lambda i: (i, 0)))

        def body(h, bi, x16r, tsr, wr, wtr, br, gr, ber, vr, o):
            _enc_body(0, h, bi, x16r, tsr, wr, wtr, br, gr, ber, o, vxyz_ref=vr)
    else:
        def body(h, bi, x16r, tsr, wr, wtr, br, gr, ber, o):
            _enc_body(0, h, bi, x16r, tsr, wr, wtr, br, gr, ber, o)
    return pl.pallas_call(
        body,
        grid=(nblk,),
        in_specs=specs,
        out_specs=pl.BlockSpec((NB, 128), lambda i: (i, 0)),
        out_shape=jax.ShapeDtypeStruct((npad, 128), jnp.float32),
    )(*ins)


def _msg_body(src_ref, dst_ref, *rest):
    out_ref = rest[-1]
    ws = _unpack_ws(rest[:-1], 3)
    src = src_ref[...]
    dst = dst_ref[...]
    s = jnp.concatenate([src[:, :64], dst[:, :64]], axis=1)
    dx = dst[:, 112:113] - src[:, 112:113]
    dy = dst[:, 113:114] - src[:, 113:114]
    dz = dst[:, 114:115] - src[:, 114:115]
    inv = 1.0 / (jnp.sqrt(dx * dx + dy * dy + dz * dz) + 1e-8)
    vx = jnp.concatenate([src[:, 64:80], dst[:, 64:80], dx * inv], axis=1)
    vy = jnp.concatenate([src[:, 80:96], dst[:, 80:96], dy * inv], axis=1)
    vz = jnp.concatenate([src[:, 96:112], dst[:, 96:112], dz * inv], axis=1)
    for w in ws:
        s, vx, vy, vz = _gvp(s, vx, vy, vz, w, gate=True)
    out_ref[...] = jnp.concatenate(
        [s, vx, vy, vz, jnp.zeros((s.shape[0], 16), jnp.float32)], axis=1)


def _messages(src_rows, dst_rows, chain):
    ep = src_rows.shape[0]
    flat = _flatten_chain(chain)
    ins = [src_rows, dst_rows] + flat
    specs = [pl.BlockSpec((EB, 128), lambda i: (i, 0)),
             pl.BlockSpec((EB, 128), lambda i: (i, 0))] + [_rep_spec(a) for a in flat]
    return pl.pallas_call(
        _msg_body,
        grid=(ep // EB,),
        in_specs=specs,
        out_specs=pl.BlockSpec((EB, MW), lambda i: (i, 0)),
        out_shape=jax.ShapeDtypeStruct((ep, MW), jnp.float32),
    )(*ins)


def _upd_body(tab_ref, agg_ref, *rest):
    out_ref = rest[-1]
    g_ref, b_ref = rest[-3], rest[-2]
    ws = _unpack_ws(rest[:-3], 2)
    tab = tab_ref[...]
    agg = agg_ref[...]
    s0 = tab[:, :64]
    vx0, vy0, vz0 = tab[:, 64:80], tab[:, 80:96], tab[:, 96:112]
    s = jnp.concatenate([s0, agg[:, :64]], axis=1)
    vx = jnp.concatenate([vx0, agg[:, 64:80]], axis=1)
    vy = jnp.concatenate([vy0, agg[:, 80:96]], axis=1)
    vz = jnp.concatenate([vz0, agg[:, 96:112]], axis=1)
    for w in ws:
        s, vx, vy, vz = _gvp(s, vx, vy, vz, w, gate=True)
    s = s0 + s
    mu = jnp.mean(s, axis=-1, keepdims=True)
    var = jnp.mean((s - mu) ** 2, axis=-1, keepdims=True)
    s = (s - mu) / jnp.sqrt(var + 1e-5) * g_ref[...] + b_ref[...]
    out_ref[...] = jnp.concatenate(
        [s, vx0 + vx, vy0 + vy, vz0 + vz, tab[:, 112:128]], axis=1)


def _update(table, agg, chain, ln_g, ln_b):
    npad = table.shape[0]
    flat = _flatten_chain(chain)
    ins = [table, agg] + flat + [ln_g.reshape(1, H), ln_b.reshape(1, H)]
    specs = ([pl.BlockSpec((NB, 128), lambda i: (i, 0)),
              pl.BlockSpec((NB, MW), lambda i: (i, 0))]
             + [_rep_spec(a) for a in flat]
             + [_rep_spec(ln_g.reshape(1, H)), _rep_spec(ln_b.reshape(1, H))])
    return pl.pallas_call(
        _upd_body,
        grid=(npad // NB,),
        in_specs=specs,
        out_specs=pl.BlockSpec((NB, 128), lambda i: (i, 0)),
        out_shape=jax.ShapeDtypeStruct((npad, 128), jnp.float32),
    )(*ins)


def _noise_body(tab_ref, *rest):
    eps_ref, v_ref = rest[-2], rest[-1]
    ow_ref, ob_ref = rest[-4], rest[-3]
    ws = _unpack_ws(rest[:-4], 3)
    tab = tab_ref[...]
    s = tab[:, :64]
    vx, vy, vz = tab[:, 64:80], tab[:, 80:96], tab[:, 96:112]
    s, vx, vy, vz = _gvp(s, vx, vy, vz, ws[0], gate=True)
    s, vx, vy, vz = _gvp(s, vx, vy, vz, ws[1], gate=True)
    s, vx, vy, vz = _gvp(s, vx, vy, vz, ws[2], gate=False)
    eps_ref[...] = jnp.dot(s, ow_ref[...], preferred_element_type=jnp.float32) + ob_ref[...]
    v_ref[...] = jnp.concatenate(
        [vx, vy, vz, jnp.zeros((NB, 5), jnp.float32)], axis=1)


def _noise_head(table, noise_chain, out_W, out_b):
    npad = table.shape[0]
    flat = _flatten_chain(noise_chain)
    ins = [table] + flat + [out_W, out_b.reshape(1, -1)]
    specs = ([pl.BlockSpec((NB, 128), lambda i: (i, 0))]
             + [_rep_spec(a) for a in flat]
             + [_rep_spec(out_W), _rep_spec(out_b.reshape(1, -1))])
    return pl.pallas_call(
        _noise_body,
        grid=(npad // NB,),
        in_specs=specs,
        out_specs=[pl.BlockSpec((NB, 64), lambda i: (i, 0)),
                   pl.BlockSpec((NB, 8), lambda i: (i, 0))],
        out_shape=[jax.ShapeDtypeStruct((npad, 64), jnp.float32),
                   jax.ShapeDtypeStruct((npad, 8), jnp.float32)],
    )(*ins)


SC_NC = 2
SC_NS = 16
SC_CH = 128
WIN = 14080
NW_SC = 4
NP_SC = WIN * NW_SC


def _sc_gather(table, idx):
    from jax.experimental.pallas import tpu_sc as plsc
    ep = idx.shape[0]
    per_w = ep // (SC_NC * SC_NS)
    iters = per_w // SC_CH
    mesh = plsc.VectorSubcoreMesh(core_axis_name="c", subcore_axis_name="s")

    @functools.partial(
        pl.kernel, mesh=mesh,
        out_type=jax.ShapeDtypeStruct((ep, 128), jnp.float32),
        scratch_types=[
            pltpu.VMEM((SC_CH,), jnp.int32),
            pltpu.VMEM((SC_CH, 128), jnp.float32),
            pltpu.SemaphoreType.DMA,
        ],
    )
    def gk(table_hbm, idx_hbm, out_hbm, idx_v, rows_v, sem):
        wid = jax.lax.axis_index("s") * SC_NC + jax.lax.axis_index("c")
        base0 = wid * per_w

        def body(i, _):
            base = base0 + i * SC_CH
            pltpu.sync_copy(idx_hbm.at[pl.ds(base, SC_CH)], idx_v)
            pltpu.async_copy(table_hbm.at[idx_v], rows_v, sem).wait()
            pltpu.sync_copy(rows_v, out_hbm.at[pl.ds(base, SC_CH)])
            return 0

        jax.lax.fori_loop(0, iters, body, 0)

    return gk(table, idx)


def _sc_scatter(msg_ll, dstm_ll, msg_kl, dstm_kl, zeros_win):
    from jax.experimental.pallas import tpu_sc as plsc
    mesh = plsc.VectorSubcoreMesh(core_axis_name="c", subcore_axis_name="s")
    per_ll = EP_LL // SC_NS
    per_kl = EP_KL // SC_NS
    rows_w = WIN // SC_NS

    @functools.partial(
        pl.kernel, mesh=mesh,
        out_type=jax.ShapeDtypeStruct((NP_SC, MW), jnp.float32),
        scratch_types=[
            pltpu.VMEM_SHARED((WIN + 8, MW), jnp.float32),
            pltpu.VMEM((SC_CH,), jnp.int32),
            pltpu.VMEM((SC_CH, MW), jnp.float32),
        ],
    )
    def sk(mll_hbm, dll_hbm, mkl_hbm, dkl_hbm, z_hbm, out_hbm,
           acc, idx_v, msg_v):
        c = jax.lax.axis_index("c")
        s = jax.lax.axis_index("s")

        def scan_edges(d_hbm, m_hbm, ep, per_sub, w):
            def body(i, _):
                base = s * per_sub + i * SC_CH
                pltpu.sync_copy(d_hbm.at[pl.ds(w * ep + base, SC_CH)], idx_v)
                pltpu.sync_copy(m_hbm.at[pl.ds(base, SC_CH)], msg_v)
                pltpu.sync_copy(msg_v, acc.at[idx_v], add=True)
                return 0

            jax.lax.fori_loop(0, per_sub // SC_CH, body, 0)

        for k in range(NW_SC // SC_NC):
            w = c + SC_NC * k
            w_lo = w * WIN
            pltpu.sync_copy(z_hbm.at[pl.ds(s * rows_w, rows_w)],
                            acc.at[pl.ds(s * rows_w, rows_w)])
            plsc.subcore_barrier()
            scan_edges(dll_hbm, mll_hbm, EP_LL, per_ll, w)
            scan_edges(dkl_hbm, mkl_hbm, EP_KL, per_kl, w)
            plsc.subcore_barrier()
            pltpu.sync_copy(acc.at[pl.ds(s * rows_w, rows_w)],
                            out_hbm.at[pl.ds(w_lo + s * rows_w, rows_w)])
            plsc.subcore_barrier()

    return sk(msg_ll, dstm_ll, msg_kl, dstm_kl, zeros_win)


def _window_idx(dst, ep):
    pad = jnp.full((ep - dst.shape[0],), -1, jnp.int32)
    d = jnp.concatenate([dst, pad])
    outs = []
    for w in range(NW_SC):
        local = d - w * WIN
        ok = (local >= 0) & (local < WIN)
        outs.append(jnp.where(ok, local, WIN))
    return jnp.concatenate(outs)


def _gather_rows(table, idx):
    return jnp.take(table, idx, axis=0)


def _segment_add(msg_ll, dst_ll, msg_kl, dst_kl, npad):
    agg = jnp.zeros((npad, MW), jnp.float32)
    agg = agg.at[dst_ll].add(msg_ll[:E_LL])
    agg = agg.at[dst_kl].add(msg_kl[:E_KL])
    return agg


def _pad_rows(a, npad):
    return jnp.concatenate(
        [a, jnp.zeros((npad - a.shape[0],) + a.shape[1:], a.dtype)], axis=0)


def kernel(lig_h0, lig_x0, kp_h0, kp_x0, kp_v0, timestep, lig_batch_idx,
           kp_batch_idx, ll_edge_index, kl_src, kl_dst, params):
    ts_pad = jnp.concatenate(
        [timestep, jnp.zeros((256 - timestep.shape[0],), jnp.float32)]).reshape(256, 1)

    lig_x16 = _pad_rows(jnp.pad(lig_x0, ((0, 0), (0, 13))), NP_L)
    kp_x16 = _pad_rows(jnp.pad(kp_x0, ((0, 0), (0, 13))), NP_K)
    kp_vxyz = _pad_rows(
        jnp.concatenate([kp_v0[..., 0], kp_v0[..., 1], kp_v0[..., 2]], axis=1), NP_K)

    lig_tab = _encode(_pad_rows(lig_h0, NP_L),
                      _pad_rows(lig_batch_idx.reshape(-1, 1), NP_L), lig_x16,
                      ts_pad, params['lig_enc_W'], params['lig_enc_b'],
                      params['lig_ln_g'], params['lig_ln_b'], None, NP_L)
    kp_tab = _encode(_pad_rows(kp_h0, NP_K),
                     _pad_rows(kp_batch_idx.reshape(-1, 1), NP_K), kp_x16,
                     ts_pad, params['kp_enc_W'], params['kp_enc_b'],
                     params['kp_ln_g'], params['kp_ln_b'], kp_vxyz, NP_K)

    ll_src = jnp.concatenate([ll_edge_index[0], jnp.zeros((EP_LL - E_LL,), jnp.int32)])
    ll_dst = jnp.concatenate([ll_edge_index[1], jnp.zeros((EP_LL - E_LL,), jnp.int32)])
    kl_srcp = jnp.concatenate([kl_src, jnp.zeros((EP_KL - E_KL,), jnp.int32)])
    kl_dstp = jnp.concatenate([kl_dst, jnp.zeros((EP_KL - E_KL,), jnp.int32)])
    dstm_ll = _window_idx(ll_edge_index[1], EP_LL)
    dstm_kl = _window_idx(kl_dst, EP_KL)
    zeros_win = jnp.zeros((WIN, MW), jnp.float32)

    for lp in params['convs']:
        src_rows = _sc_gather(lig_tab, ll_src)
        dst_rows = _sc_gather(lig_tab, ll_dst)
        msg_ll = _messages(src_rows, dst_rows, lp['ll_msg'])
        src2 = _sc_gather(kp_tab, kl_srcp)
        dst2 = _sc_gather(lig_tab, kl_dstp)
        msg_kl = _messages(src2, dst2, lp['kl_msg'])
        agg = _sc_scatter(msg_ll, dstm_ll, msg_kl, dstm_kl, zeros_win)
        lig_tab = _update(lig_tab, agg, lp['upd'], lp['ln_g'], lp['ln_b'])

    eps_p, v_p = _noise_head(lig_tab, params['noise'], params['out_W'],
                             params['out_b'])
    eps_h = eps_p[:NLIG]
    v = jnp.stack([v_p[:NLIG, 0], v_p[:NLIG, 1], v_p[:NLIG, 2]], axis=1).reshape(NLIG, 1, 3)
    return eps_h, v
